# baseline (speedup 1.0000x reference)
import functools

import jax
import jax.numpy as jnp
from jax import lax
from jax.experimental import pallas as pl
from jax.experimental.pallas import tpu as pltpu
from jax.experimental.pallas import tpu_sc as plsc

_H = 64
_B = 16384
_NC = 2
_NS = 16
_NW = _NC * _NS
_BPW = _B // _NW
_CHUNK = 128
_NCHUNK = _BPW // _CHUNK


def _make_gather():
    mesh = plsc.VectorSubcoreMesh(core_axis_name="c", subcore_axis_name="s")

    @functools.partial(
        pl.kernel,
        mesh=mesh,
        out_type=jax.ShapeDtypeStruct((_B, _H), jnp.float32),
        compiler_params=pltpu.CompilerParams(use_tc_tiling_on_sc=False),
        scratch_types=[
            pltpu.VMEM((_NCHUNK, _CHUNK), jnp.int32),
            pltpu.VMEM((_BPW, _H), jnp.float32),
            pltpu.SemaphoreType.DMA,
        ],
    )
    def k(table_hbm, idx_hbm, out_hbm, idx_v, rows_v, sem):
        wid = lax.axis_index("s") * _NC + lax.axis_index("c")
        pltpu.sync_copy(idx_hbm.at[wid], idx_v)
        copies = [
            pltpu.async_copy(
                table_hbm.at[idx_v.at[c]],
                rows_v.at[pl.ds(c * _CHUNK, _CHUNK)],
                sem,
            )
            for c in range(_NCHUNK)
        ]
        for cp in copies:
            cp.wait()
        pltpu.sync_copy(rows_v, out_hbm.at[pl.ds(wid * _BPW, _BPW)])

    return k


_gather = _make_gather()


def kernel(h, embedding_weight):
    idx = h.reshape(_NW, _NCHUNK, _CHUNK)
    return _gather(embedding_weight, idx)


# speedup vs baseline: 1.6284x; 1.6284x over previous
import functools

import jax
import jax.numpy as jnp
from jax import lax
from jax.experimental import pallas as pl
from jax.experimental.pallas import tpu as pltpu
from jax.experimental.pallas import tpu_sc as plsc

_H = 64
_B = 16384
_NC = 2
_NS = 16
_NW = _NC * _NS
_BPW = _B // _NW
_K = 16
_NGRP = _BPW // _K


def _make_gather():
    mesh = plsc.VectorSubcoreMesh(core_axis_name="c", subcore_axis_name="s")

    @functools.partial(
        pl.kernel,
        mesh=mesh,
        out_type=jax.ShapeDtypeStruct((_B, _H), jnp.float32),
        scratch_types=[
            pltpu.VMEM((_BPW,), jnp.int32),
            pltpu.VMEM((_BPW, _H), jnp.float32),
            pltpu.SemaphoreType.DMA,
        ],
    )
    def k(table_hbm, idx_hbm, out_hbm, idx_v, rows_v, sem):
        wid = lax.axis_index("s") * _NC + lax.axis_index("c")
        pltpu.sync_copy(idx_hbm.at[wid], idx_v)

        def body(j, carry):
            iv = idx_v[pl.ds(j * _K, _K)]
            copies = []
            for t in range(_K):
                copies.append(
                    pltpu.async_copy(table_hbm.at[iv[t]], rows_v.at[j * _K + t], sem)
                )
            for cp in copies:
                cp.wait()
            return carry

        lax.fori_loop(0, _NGRP, body, 0)
        pltpu.sync_copy(rows_v, out_hbm.at[pl.ds(wid * _BPW, _BPW)])

    return k


_gather = _make_gather()


def kernel(h, embedding_weight):
    idx = h.reshape(_NW, _BPW)
    return _gather(embedding_weight, idx)


# speedup vs baseline: 2.8612x; 1.7570x over previous
import functools

import jax
import jax.numpy as jnp
from jax import lax
from jax.experimental import pallas as pl
from jax.experimental.pallas import tpu as pltpu
from jax.experimental.pallas import tpu_sc as plsc

_N = 1000000
_H = 64
_B = 16384
_NC = 2
_NS = 16
_NW = _NC * _NS
_BPW = _B // _NW
_FPW = _BPW * _H
_K = 16
_NGRP = _BPW // _K
_R = 8
_LB = 128
_TAIL = (_N // _LB) * _LB


def _make_gather():
    mesh = plsc.VectorSubcoreMesh(core_axis_name="c", subcore_axis_name="s")

    @functools.partial(
        pl.kernel,
        mesh=mesh,
        out_type=jax.ShapeDtypeStruct((_B * _H,), jnp.float32),
        compiler_params=pltpu.CompilerParams(needs_layout_passes=False),
        scratch_types=[
            pltpu.VMEM((_BPW,), jnp.int32),
            pltpu.VMEM((_R, _H, _LB), jnp.float32),
            pltpu.VMEM((_FPW,), jnp.float32),
            pltpu.SemaphoreType.DMA((_R,)),
        ],
    )
    def k(table_t_hbm, tail_hbm, idx_hbm, out_hbm, idx_v, blk_v, rows_v, sems):
        wid = lax.axis_index("s") * _NC + lax.axis_index("c")
        pltpu.sync_copy(idx_hbm.at[wid], idx_v)
        jcol = lax.iota(jnp.int32, 16)

        def fire(m, i):
            ib = (i >> 7) * _LB

            @pl.when(i < _TAIL)
            def _():
                pltpu.async_copy(
                    table_t_hbm.at[:, pl.ds(ib, _LB)], blk_v.at[m % _R],
                    sems.at[m % _R],
                )

            @pl.when(i >= _TAIL)
            def _():
                pltpu.async_copy(
                    tail_hbm, blk_v.at[m % _R], sems.at[m % _R],
                )

        def extract(m, i):
            li = jnp.full((16,), i & (_LB - 1), jnp.int32)
            pltpu.make_async_copy(
                table_t_hbm.at[:, pl.ds(0, _LB)], blk_v.at[m % _R],
                sems.at[m % _R],
            ).wait()
            for g in range(4):
                vals = plsc.load_gather(blk_v.at[m % _R], [jcol + 16 * g, li])
                rows_v[pl.ds(m * _H + 16 * g, 16)] = vals

        def body(j, carry):
            iv = idx_v[pl.ds(j * _K, _K)]
            ivp = idx_v[pl.ds(jnp.maximum(j - 1, 0) * _K, _K)]
            for t in range(8):
                m = j * _K + t

                @pl.when(j >= 1)
                def _(m=m, t=t):
                    extract(m - _R, ivp[t + 8])

                fire(m, iv[t])
            for t in range(8, 16):
                m = j * _K + t
                extract(m - _R, iv[t - 8])
                fire(m, iv[t])
            return carry

        lax.fori_loop(0, _NGRP, body, 0)
        iv_last = idx_v[pl.ds((_NGRP - 1) * _K, _K)]
        for t in range(8, 16):
            m = (_NGRP - 1) * _K + t
            extract(m, iv_last[t])

        pltpu.sync_copy(rows_v, out_hbm.at[pl.ds(wid * _FPW, _FPW)])

    return k


_gather = _make_gather()


def kernel(h, embedding_weight):
    idx = h.reshape(_NW, _BPW)
    tail = jnp.pad(
        embedding_weight[_TAIL:].T, ((0, 0), (0, _LB - (_N - _TAIL)))
    )
    out = _gather(embedding_weight.T, tail, idx)
    return out.reshape(_B, _H)


# speedup vs baseline: 2.8643x; 1.0011x over previous
import functools

import jax
import jax.numpy as jnp
from jax import lax
from jax.experimental import pallas as pl
from jax.experimental.pallas import tpu as pltpu
from jax.experimental.pallas import tpu_sc as plsc

_N = 1000000
_H = 64
_B = 16384
_NC = 2
_NS = 16
_NW = _NC * _NS
_BPW = _B // _NW
_FPW = _BPW * _H
_K = 16
_NGRP = _BPW // _K
_R = 8
_LB = 128
_TAIL = (_N // _LB) * _LB


def _make_gather():
    mesh = plsc.VectorSubcoreMesh(core_axis_name="c", subcore_axis_name="s")

    @functools.partial(
        pl.kernel,
        mesh=mesh,
        out_type=jax.ShapeDtypeStruct((_B * _H,), jnp.float32),
        compiler_params=pltpu.CompilerParams(needs_layout_passes=False),
        scratch_types=[
            pltpu.VMEM((_BPW,), jnp.int32),
            pltpu.VMEM((_R, _H, _LB), jnp.float32),
            pltpu.VMEM((_FPW,), jnp.float32),
            pltpu.SemaphoreType.DMA((_R,)),
        ],
    )
    def k(table_t_hbm, tail_hbm, idx_hbm, out_hbm, idx_v, blk_v, rows_v, sems):
        wid = lax.axis_index("s") * _NC + lax.axis_index("c")
        pltpu.sync_copy(idx_hbm.at[wid], idx_v)
        jcol = lax.iota(jnp.int32, 16)

        def fire(m, i):
            ib = (i >> 7) * _LB

            @pl.when(i < _TAIL)
            def _():
                pltpu.async_copy(
                    table_t_hbm.at[:, pl.ds(ib, _LB)], blk_v.at[m % _R],
                    sems.at[m % _R],
                )

            @pl.when(i >= _TAIL)
            def _():
                pltpu.async_copy(
                    tail_hbm, blk_v.at[m % _R], sems.at[m % _R],
                )

        def extract(m, i):
            li = jnp.full((16,), i & (_LB - 1), jnp.int32)
            pltpu.make_async_copy(
                table_t_hbm.at[:, pl.ds(0, _LB)], blk_v.at[m % _R],
                sems.at[m % _R],
            ).wait()
            for g in range(4):
                vals = plsc.load_gather(blk_v.at[m % _R], [jcol + 16 * g, li])
                rows_v[pl.ds(m * _H + 16 * g, 16)] = vals

        _LAG = _R - 1

        def body(j, carry):
            iv = idx_v[pl.ds(j * _K, _K)]
            ivp = idx_v[pl.ds(jnp.maximum(j - 1, 0) * _K, _K)]
            for t in range(_K):
                m = j * _K + t
                fire(m, iv[t])
                if t < _LAG:

                    @pl.when(j >= 1)
                    def _(m=m, t=t):
                        extract(m - _LAG, ivp[t + _K - _LAG])

                else:
                    extract(m - _LAG, iv[t - _LAG])
            return carry

        lax.fori_loop(0, _NGRP, body, 0)
        iv_last = idx_v[pl.ds((_NGRP - 1) * _K, _K)]
        for t in range(_K - _LAG, _K):
            m = (_NGRP - 1) * _K + t
            extract(m, iv_last[t])

        pltpu.sync_copy(rows_v, out_hbm.at[pl.ds(wid * _FPW, _FPW)])

    return k


_gather = _make_gather()


def kernel(h, embedding_weight):
    idx = h.reshape(_NW, _BPW)
    tail = jnp.pad(
        embedding_weight[_TAIL:].T, ((0, 0), (0, _LB - (_N - _TAIL)))
    )
    out = _gather(embedding_weight.T, tail, idx)
    return out.reshape(_B, _H)


# speedup vs baseline: 2.8692x; 1.0017x over previous
import functools

import jax
import jax.numpy as jnp
from jax import lax
from jax.experimental import pallas as pl
from jax.experimental.pallas import tpu as pltpu
from jax.experimental.pallas import tpu_sc as plsc

_N = 1000000
_H = 64
_B = 16384
_NC = 2
_NS = 16
_NW = _NC * _NS
_BPW = _B // _NW
_FPW = _BPW * _H
_K = 16
_NGRP = _BPW // _K
_R = 8
_LB = 128
_TAIL = (_N // _LB) * _LB


def _make_gather():
    mesh = plsc.VectorSubcoreMesh(core_axis_name="c", subcore_axis_name="s")

    @functools.partial(
        pl.kernel,
        mesh=mesh,
        out_type=jax.ShapeDtypeStruct((_B * _H,), jnp.float32),
        compiler_params=pltpu.CompilerParams(needs_layout_passes=False),
        scratch_types=[
            pltpu.VMEM((_BPW,), jnp.int32),
            pltpu.VMEM((_R, _H, _LB), jnp.float32),
            pltpu.VMEM((_FPW,), jnp.float32),
            pltpu.SemaphoreType.DMA((_R,)),
        ],
    )
    def k(table_t_hbm, tail_hbm, idx_hbm, out_hbm, idx_v, blk_v, rows_v, sems):
        wid = lax.axis_index("s") * _NC + lax.axis_index("c")
        pltpu.sync_copy(idx_hbm.at[pl.ds(wid * _BPW, _BPW)], idx_v)
        jcol = lax.iota(jnp.int32, 16)

        def fire(m, i):
            ib = (i >> 7) * _LB

            @pl.when(i < _TAIL)
            def _():
                pltpu.async_copy(
                    table_t_hbm.at[:, pl.ds(ib, _LB)], blk_v.at[m % _R],
                    sems.at[m % _R],
                )

            @pl.when(i >= _TAIL)
            def _():
                pltpu.async_copy(
                    tail_hbm, blk_v.at[m % _R], sems.at[m % _R],
                )

        def extract(m, i):
            li = jnp.full((16,), i & (_LB - 1), jnp.int32)
            pltpu.make_async_copy(
                table_t_hbm.at[:, pl.ds(0, _LB)], blk_v.at[m % _R],
                sems.at[m % _R],
            ).wait()
            for g in range(4):
                vals = plsc.load_gather(blk_v.at[m % _R], [jcol + 16 * g, li])
                rows_v[pl.ds(m * _H + 16 * g, 16)] = vals

        _LAG = _R - 1

        def body(j, carry):
            iv = idx_v[pl.ds(j * _K, _K)]
            ivp = idx_v[pl.ds(jnp.maximum(j - 1, 0) * _K, _K)]
            for t in range(_K):
                m = j * _K + t
                fire(m, iv[t])
                if t < _LAG:

                    @pl.when(j >= 1)
                    def _(m=m, t=t):
                        extract(m - _LAG, ivp[t + _K - _LAG])

                else:
                    extract(m - _LAG, iv[t - _LAG])
            return carry

        lax.fori_loop(0, _NGRP, body, 0)
        iv_last = idx_v[pl.ds((_NGRP - 1) * _K, _K)]
        for t in range(_K - _LAG, _K):
            m = (_NGRP - 1) * _K + t
            extract(m, iv_last[t])

        pltpu.sync_copy(rows_v, out_hbm.at[pl.ds(wid * _FPW, _FPW)])

    return k


_gather = _make_gather()


def kernel(h, embedding_weight):
    idx = h.reshape(_B)
    tail = jnp.pad(
        embedding_weight[_TAIL:].T, ((0, 0), (0, _LB - (_N - _TAIL)))
    )
    out = _gather(embedding_weight.T, tail, idx)
    return out.reshape(_B, _H)


# speedup vs baseline: 2.9349x; 1.0229x over previous
import functools

import jax
import jax.numpy as jnp
from jax import lax
from jax.experimental import pallas as pl
from jax.experimental.pallas import tpu as pltpu
from jax.experimental.pallas import tpu_sc as plsc

_N = 1000000
_H = 64
_B = 16384
_NC = 2
_NS = 16
_NW = _NC * _NS
_BPW = _B // _NW
_FPW = _BPW * _H
_K = 16
_NGRP = _BPW // _K
_R = 10
_LB = 128
_TAIL = (_N // _LB) * _LB


def _make_gather():
    mesh = plsc.VectorSubcoreMesh(core_axis_name="c", subcore_axis_name="s")

    @functools.partial(
        pl.kernel,
        mesh=mesh,
        out_type=jax.ShapeDtypeStruct((_B * _H,), jnp.float32),
        compiler_params=pltpu.CompilerParams(needs_layout_passes=False),
        scratch_types=[
            pltpu.VMEM((_BPW,), jnp.int32),
            pltpu.VMEM((_R, _H, _LB), jnp.float32),
            pltpu.VMEM((_FPW,), jnp.float32),
            pltpu.SemaphoreType.DMA((_R,)),
        ],
    )
    def k(table_t_hbm, tail_hbm, idx_hbm, out_hbm, idx_v, blk_v, rows_v, sems):
        wid = lax.axis_index("s") * _NC + lax.axis_index("c")
        pltpu.sync_copy(idx_hbm.at[pl.ds(wid * _BPW, _BPW)], idx_v)
        jcol = lax.iota(jnp.int32, 16)

        def fire(m, i):
            ib = (i >> 7) * _LB

            @pl.when(i < _TAIL)
            def _():
                pltpu.async_copy(
                    table_t_hbm.at[:, pl.ds(ib, _LB)], blk_v.at[m % _R],
                    sems.at[m % _R],
                )

            @pl.when(i >= _TAIL)
            def _():
                pltpu.async_copy(
                    tail_hbm, blk_v.at[m % _R], sems.at[m % _R],
                )

        def extract(m, i):
            li = jnp.full((16,), i & (_LB - 1), jnp.int32)
            pltpu.make_async_copy(
                table_t_hbm.at[:, pl.ds(0, _LB)], blk_v.at[m % _R],
                sems.at[m % _R],
            ).wait()
            for g in range(4):
                vals = plsc.load_gather(blk_v.at[m % _R], [jcol + 16 * g, li])
                rows_v[pl.ds(m * _H + 16 * g, 16)] = vals

        _LAG = _R - 1

        def body(j, carry):
            iv = idx_v[pl.ds(j * _K, _K)]
            ivp = idx_v[pl.ds(jnp.maximum(j - 1, 0) * _K, _K)]
            for t in range(_K):
                m = j * _K + t
                fire(m, iv[t])
                if t < _LAG:

                    @pl.when(j >= 1)
                    def _(m=m, t=t):
                        extract(m - _LAG, ivp[t + _K - _LAG])

                else:
                    extract(m - _LAG, iv[t - _LAG])
            return carry

        lax.fori_loop(0, _NGRP, body, 0)
        iv_last = idx_v[pl.ds((_NGRP - 1) * _K, _K)]
        for t in range(_K - _LAG, _K):
            m = (_NGRP - 1) * _K + t
            extract(m, iv_last[t])

        pltpu.sync_copy(rows_v, out_hbm.at[pl.ds(wid * _FPW, _FPW)])

    return k


_gather = _make_gather()


def kernel(h, embedding_weight):
    idx = h.reshape(_B)
    tail = jnp.pad(
        embedding_weight[_TAIL:].T, ((0, 0), (0, _LB - (_N - _TAIL)))
    )
    out = _gather(embedding_weight.T, tail, idx)
    return out.reshape(_B, _H)


# speedup vs baseline: 2.9451x; 1.0035x over previous
import functools

import jax
import jax.numpy as jnp
from jax import lax
from jax.experimental import pallas as pl
from jax.experimental.pallas import tpu as pltpu
from jax.experimental.pallas import tpu_sc as plsc

_N = 1000000
_H = 64
_B = 16384
_NC = 2
_NS = 16
_NW = _NC * _NS
_BPW = _B // _NW
_FPW = _BPW * _H
_K = 16
_NGRP = _BPW // _K
_R = 11
_LB = 128
_TAIL = (_N // _LB) * _LB


def _make_gather():
    mesh = plsc.VectorSubcoreMesh(core_axis_name="c", subcore_axis_name="s")

    @functools.partial(
        pl.kernel,
        mesh=mesh,
        out_type=jax.ShapeDtypeStruct((_B * _H,), jnp.float32),
        compiler_params=pltpu.CompilerParams(needs_layout_passes=False),
        scratch_types=[
            pltpu.VMEM((_BPW,), jnp.int32),
            pltpu.VMEM((_R, _H, _LB), jnp.float32),
            pltpu.VMEM((_FPW,), jnp.float32),
            pltpu.SemaphoreType.DMA((_R,)),
        ],
    )
    def k(table_t_hbm, tail_hbm, idx_hbm, out_hbm, idx_v, blk_v, rows_v, sems):
        wid = lax.axis_index("s") * _NC + lax.axis_index("c")
        pltpu.sync_copy(idx_hbm.at[pl.ds(wid * _BPW, _BPW)], idx_v)
        jcol = lax.iota(jnp.int32, 16)

        def fire(m, i):
            ib = (i >> 7) * _LB

            @pl.when(i < _TAIL)
            def _():
                pltpu.async_copy(
                    table_t_hbm.at[:, pl.ds(ib, _LB)], blk_v.at[m % _R],
                    sems.at[m % _R],
                )

            @pl.when(i >= _TAIL)
            def _():
                pltpu.async_copy(
                    tail_hbm, blk_v.at[m % _R], sems.at[m % _R],
                )

        def extract(m, i):
            li = jnp.full((16,), i & (_LB - 1), jnp.int32)
            pltpu.make_async_copy(
                table_t_hbm.at[:, pl.ds(0, _LB)], blk_v.at[m % _R],
                sems.at[m % _R],
            ).wait()
            for g in range(4):
                vals = plsc.load_gather(blk_v.at[m % _R], [jcol + 16 * g, li])
                rows_v[pl.ds(m * _H + 16 * g, 16)] = vals

        _LAG = _R - 1

        def body(j, carry):
            iv = idx_v[pl.ds(j * _K, _K)]
            ivp = idx_v[pl.ds(jnp.maximum(j - 1, 0) * _K, _K)]
            for t in range(_K):
                m = j * _K + t
                fire(m, iv[t])
                if t < _LAG:

                    @pl.when(j >= 1)
                    def _(m=m, t=t):
                        extract(m - _LAG, ivp[t + _K - _LAG])

                else:
                    extract(m - _LAG, iv[t - _LAG])
            return carry

        lax.fori_loop(0, _NGRP, body, 0)
        iv_last = idx_v[pl.ds((_NGRP - 1) * _K, _K)]
        for t in range(_K - _LAG, _K):
            m = (_NGRP - 1) * _K + t
            extract(m, iv_last[t])

        pltpu.sync_copy(rows_v, out_hbm.at[pl.ds(wid * _FPW, _FPW)])

    return k


_gather = _make_gather()


def kernel(h, embedding_weight):
    idx = h.reshape(_B)
    tail = jnp.pad(
        embedding_weight[_TAIL:].T, ((0, 0), (0, _LB - (_N - _TAIL)))
    )
    out = _gather(embedding_weight.T, tail, idx)
    return out.reshape(_B, _H)
